# baseline (speedup 1.0000x reference)
import functools

import jax
import jax.numpy as jnp
from jax import lax
from jax.experimental import pallas as pl
from jax.experimental.pallas import tpu as pltpu
from jax.experimental.pallas import tpu_sc as plsc

AX0, AX1 = 64, 128
D0, D1 = 256, 768
DH = D0 + D1

NC, NS, L = 2, 16, 16
NW = NC * NS


def _make_sc_kernel(n):
    pw = n // NW
    g = 4
    nchunk = pw // g
    mesh = plsc.VectorSubcoreMesh(core_axis_name="c", subcore_axis_name="s")

    @functools.partial(
        pl.kernel,
        mesh=mesh,
        compiler_params=pltpu.CompilerParams(needs_layout_passes=False),
        out_type=jax.ShapeDtypeStruct((n * DH,), jnp.float32),
        scratch_types=[
            pltpu.VMEM((pw,), jnp.int32),
            pltpu.VMEM((pw,), jnp.int32),
            pltpu.VMEM((pw,), jnp.int32),
            pltpu.VMEM((AX0 * D0,), jnp.float32),
            pltpu.VMEM((AX1 * D1,), jnp.float32),
            pltpu.VMEM((2, g * DH), jnp.float32),
            pltpu.SemaphoreType.DMA,
            pltpu.SemaphoreType.DMA,
        ],
    )
    def kern(pos_hbm, w0_hbm, w1_hbm, out_hbm,
             pos_v, a0_v, a1_v, w0t, w1t, stage, gsem, wsem):
        wid = lax.axis_index("s") * NC + lax.axis_index("c")
        base = wid * pw
        pltpu.sync_copy(pos_hbm.at[pl.ds(base, pw)], pos_v)
        tw0 = pltpu.async_copy(w0_hbm, w0t, gsem)
        tw1 = pltpu.async_copy(w1_hbm, w1t, gsem)
        for i in range(pw // L):
            p16 = pos_v[pl.ds(i * L, L)]
            a0_v[pl.ds(i * L, L)] = lax.shift_left(
                lax.shift_right_logical(p16, 7), 8)
            a1_v[pl.ds(i * L, L)] = lax.bitwise_and(p16, 127) * 768
        tw0.wait()
        tw1.wait()

        iota = lax.iota(jnp.int32, L)

        def assemble(c, b):
            for j in range(g):
                q = c * g + j
                qv = jnp.broadcast_to(q, (L,))
                a0 = plsc.load_gather(a0_v, [qv]) + iota
                a1 = plsc.load_gather(a1_v, [qv]) + iota
                for jj in range(D0 // L):
                    stage[b, pl.ds(j * DH + jj * L, L)] = (
                        plsc.load_gather(w0t, [a0 + jj * L]))
                for jj in range(D1 // L):
                    stage[b, pl.ds(j * DH + D0 + jj * L, L)] = (
                        plsc.load_gather(w1t, [a1 + jj * L]))

        def fire_write(c, b):
            pltpu.async_copy(
                stage.at[b],
                out_hbm.at[pl.ds((base + c * g) * DH, g * DH)], wsem)

        def drain_write(c, b):
            pltpu.make_async_copy(
                stage.at[b],
                out_hbm.at[pl.ds((base + c * g) * DH, g * DH)], wsem).wait()

        def body(c0, carry):
            for b in range(2):
                c = c0 + b

                @pl.when(c0 > 0)
                def _():
                    drain_write(c - 2, b)
                assemble(c, b)
                fire_write(c, b)
            return carry

        lax.fori_loop(0, nchunk // 2, lambda i, cr: body(i * 2, cr), 0,
                      unroll=False)
        for b in range(2):
            drain_write(nchunk - 2 + b, b)

    return kern


def kernel(position_ids, w0, w1):
    b, s = position_ids.shape
    n = b * s
    pos = position_ids.reshape(n).astype(jnp.int32)
    w0f = w0.reshape(AX0 * D0)
    w1f = w1.reshape(AX1 * D1)
    out = _make_sc_kernel(n)(pos, w0f, w1f)
    return out.reshape(b, s, DH)


# speedup vs baseline: 1.9783x; 1.9783x over previous
import functools

import jax
import jax.numpy as jnp
from jax import lax
from jax.experimental import pallas as pl
from jax.experimental.pallas import tpu as pltpu
from jax.experimental.pallas import tpu_sc as plsc

AX0, AX1 = 64, 128
D0, D1 = 256, 768
DH = D0 + D1

NC, NS, L = 2, 16, 16
NW = NC * NS


def _make_sc_kernel(n):
    pw = n // NW
    g = 4
    nchunk = pw // g
    mesh = plsc.VectorSubcoreMesh(core_axis_name="c", subcore_axis_name="s")

    @functools.partial(
        pl.kernel,
        mesh=mesh,
        compiler_params=pltpu.CompilerParams(needs_layout_passes=False),
        out_type=jax.ShapeDtypeStruct((n * DH,), jnp.float32),
        scratch_types=[
            pltpu.VMEM((pw,), jnp.int32),
            pltpu.VMEM((pw,), jnp.int32),
            pltpu.VMEM((pw,), jnp.int32),
            pltpu.VMEM((AX0 * D0,), jnp.float32),
            pltpu.VMEM((AX1 * D1,), jnp.float32),
            pltpu.VMEM((2, g * DH), jnp.float32),
            pltpu.SemaphoreType.DMA,
            pltpu.SemaphoreType.DMA,
        ],
    )
    def kern(pos_hbm, w0_hbm, w1_hbm, out_hbm,
             pos_v, a0_v, a1_v, w0t, w1t, stage, gsem, wsem):
        wid = lax.axis_index("s") * NC + lax.axis_index("c")
        base = wid * pw
        pltpu.sync_copy(pos_hbm.at[pl.ds(base, pw)], pos_v)
        tw0 = pltpu.async_copy(w0_hbm, w0t, gsem)
        tw1 = pltpu.async_copy(w1_hbm, w1t, gsem)
        for i in range(pw // L):
            p16 = pos_v[pl.ds(i * L, L)]
            a0_v[pl.ds(i * L, L)] = lax.shift_left(
                lax.shift_right_logical(p16, 7), 8)
            a1_v[pl.ds(i * L, L)] = lax.bitwise_and(p16, 127) * 768
        tw0.wait()
        tw1.wait()

        iota = lax.iota(jnp.int32, L)

        def assemble(c, b):
            for j in range(g):
                q = c * g + j
                qv = jnp.broadcast_to(q, (L,))
                a0 = plsc.load_gather(a0_v, [qv]) + iota
                a1 = plsc.load_gather(a1_v, [qv]) + iota

                @plsc.parallel_loop(0, D0 // L, unroll=8)
                def _(jj, _j=j, _a0=a0):
                    stage[b, pl.ds(_j * DH + jj * L, L)] = (
                        plsc.load_gather(w0t, [_a0 + jj * L]))

                @plsc.parallel_loop(0, D1 // L, unroll=8)
                def _(jj, _j=j, _a1=a1):
                    stage[b, pl.ds(_j * DH + D0 + jj * L, L)] = (
                        plsc.load_gather(w1t, [_a1 + jj * L]))

        def fire_write(c, b):
            pltpu.async_copy(
                stage.at[b],
                out_hbm.at[pl.ds((base + c * g) * DH, g * DH)], wsem)

        def drain_write(c, b):
            pltpu.make_async_copy(
                stage.at[b],
                out_hbm.at[pl.ds((base + c * g) * DH, g * DH)], wsem).wait()

        def body(c0, carry):
            for b in range(2):
                c = c0 + b

                @pl.when(c0 > 0)
                def _():
                    drain_write(c - 2, b)
                assemble(c, b)
                fire_write(c, b)
            return carry

        lax.fori_loop(0, nchunk // 2, lambda i, cr: body(i * 2, cr), 0,
                      unroll=False)
        for b in range(2):
            drain_write(nchunk - 2 + b, b)

    return kern


def kernel(position_ids, w0, w1):
    b, s = position_ids.shape
    n = b * s
    pos = position_ids.reshape(n).astype(jnp.int32)
    w0f = w0.reshape(AX0 * D0)
    w1f = w1.reshape(AX1 * D1)
    out = _make_sc_kernel(n)(pos, w0f, w1f)
    return out.reshape(b, s, DH)


# speedup vs baseline: 2.0643x; 1.0435x over previous
import functools

import jax
import jax.numpy as jnp
from jax import lax
from jax.experimental import pallas as pl
from jax.experimental.pallas import tpu as pltpu
from jax.experimental.pallas import tpu_sc as plsc

AX0, AX1 = 64, 128
D0, D1 = 256, 768
DH = D0 + D1

NC, NS, L = 2, 16, 16
NW = NC * NS


def _make_sc_kernel(n):
    pw = n // NW
    g = 4
    nchunk = pw // g
    mesh = plsc.VectorSubcoreMesh(core_axis_name="c", subcore_axis_name="s")

    @functools.partial(
        pl.kernel,
        mesh=mesh,
        compiler_params=pltpu.CompilerParams(needs_layout_passes=False),
        out_type=jax.ShapeDtypeStruct((n * DH,), jnp.float32),
        scratch_types=[
            pltpu.VMEM((pw,), jnp.int32),
            pltpu.VMEM((pw,), jnp.int32),
            pltpu.VMEM((pw,), jnp.int32),
            pltpu.VMEM((AX0 * D0,), jnp.float32),
            pltpu.VMEM((AX1 * D1,), jnp.float32),
            pltpu.VMEM((2, g * DH), jnp.float32),
            pltpu.SemaphoreType.DMA,
            pltpu.SemaphoreType.DMA,
        ],
    )
    def kern(pos_hbm, w0_hbm, w1_hbm, out_hbm,
             pos_v, a0_v, a1_v, w0t, w1t, stage, gsem, wsem):
        wid = lax.axis_index("s") * NC + lax.axis_index("c")
        base = wid * pw
        pltpu.sync_copy(pos_hbm.at[pl.ds(base, pw)], pos_v)
        tw0 = pltpu.async_copy(w0_hbm, w0t, gsem)
        tw1 = pltpu.async_copy(w1_hbm, w1t, gsem)
        for i in range(pw // L):
            p16 = pos_v[pl.ds(i * L, L)]
            a0_v[pl.ds(i * L, L)] = lax.shift_left(
                lax.shift_right_logical(p16, 7), 8)
            a1_v[pl.ds(i * L, L)] = lax.bitwise_and(p16, 127) * 768
        tw0.wait()
        tw1.wait()

        iota = lax.iota(jnp.int32, L)

        def assemble(c, b):
            for j in range(g):
                q = c * g + j
                qv = jnp.broadcast_to(q, (L,))
                a0 = plsc.load_gather(a0_v, [qv]) + iota
                a1 = plsc.load_gather(a1_v, [qv]) + iota

                @plsc.parallel_loop(0, D0 // L, unroll=16)
                def _(jj, _j=j, _a0=a0):
                    stage[b, pl.ds(_j * DH + jj * L, L)] = (
                        plsc.load_gather(w0t, [_a0 + jj * L]))

                @plsc.parallel_loop(0, D1 // L, unroll=16)
                def _(jj, _j=j, _a1=a1):
                    stage[b, pl.ds(_j * DH + D0 + jj * L, L)] = (
                        plsc.load_gather(w1t, [_a1 + jj * L]))

        def fire_write(c, b):
            pltpu.async_copy(
                stage.at[b],
                out_hbm.at[pl.ds((base + c * g) * DH, g * DH)], wsem)

        def drain_write(c, b):
            pltpu.make_async_copy(
                stage.at[b],
                out_hbm.at[pl.ds((base + c * g) * DH, g * DH)], wsem).wait()

        def body(c0, carry):
            for b in range(2):
                c = c0 + b

                @pl.when(c0 > 0)
                def _():
                    drain_write(c - 2, b)
                assemble(c, b)
                fire_write(c, b)
            return carry

        lax.fori_loop(0, nchunk // 2, lambda i, cr: body(i * 2, cr), 0,
                      unroll=False)
        for b in range(2):
            drain_write(nchunk - 2 + b, b)

    return kern


def kernel(position_ids, w0, w1):
    b, s = position_ids.shape
    n = b * s
    pos = position_ids.reshape(n).astype(jnp.int32)
    w0f = w0.reshape(AX0 * D0)
    w1f = w1.reshape(AX1 * D1)
    out = _make_sc_kernel(n)(pos, w0f, w1f)
    return out.reshape(b, s, DH)
